# baseline (speedup 1.0000x reference)
import functools

import jax
import jax.numpy as jnp
from jax import lax
from jax.experimental import pallas as pl
from jax.experimental.pallas import tpu as pltpu
from jax.experimental.pallas import tpu_sc as plsc

N = 10000
D = 128
E = 320000

_NC = 2
_NS = 16
_NW = _NC * _NS
_C = 80
_NCHUNKS = E // _C
_PW = _NCHUNKS // _NW
_RPT = 624
_WB = 48


def _tile_ids():
    cid = lax.axis_index("c")
    sid = lax.axis_index("s")
    return cid, sid, sid * _NC + cid


def _zero_acc(acc_s, zsrc_v, sid):
    r0 = 16 + sid * _RPT
    for k in range(_RPT // _WB):
        pltpu.sync_copy(zsrc_v.at[pl.ds(0, _WB)],
                        acc_s.at[pl.ds(r0 + k * _WB, _WB)])

    @pl.when(sid < 2)
    def _zero_head():
        pltpu.sync_copy(zsrc_v.at[pl.ds(0, 8)], acc_s.at[pl.ds(sid * 8, 8)])


def _write_acc(acc_s, stage_v, out_hbm, cid, sid):
    r0 = 16 + sid * _RPT
    for k in range(_RPT // _WB):
        rs = r0 + k * _WB
        pltpu.sync_copy(acc_s.at[pl.ds(rs, _WB)], stage_v.at[pl.ds(0, _WB)])
        pltpu.sync_copy(stage_v.at[pl.ds(0, _WB)],
                        out_hbm.at[cid, pl.ds(rs, _WB)])

    @pl.when(sid < 2)
    def _write_head():
        hs = sid * 8
        pltpu.sync_copy(acc_s.at[pl.ds(hs, 8)], stage_v.at[pl.ds(0, 8)])
        pltpu.sync_copy(stage_v.at[pl.ds(0, 8)], out_hbm.at[cid, pl.ds(hs, 8)])


def _pipeline(wid, start, finish):
    def _c(i):
        return i * _NW + wid

    start(_c(0), 0)

    def _body(k, carry):
        start(_c(2 * k + 1), 1)
        finish(_c(2 * k), 0)
        start(_c(2 * k + 2), 0)
        finish(_c(2 * k + 1), 1)
        return carry
    lax.fori_loop(0, (_PW - 1) // 2, _body, 0)
    finish(_c(_PW - 1), 0)


@functools.cache
def _make_sc_aggregate():
    mesh = plsc.VectorSubcoreMesh(core_axis_name="c", subcore_axis_name="s")

    def body(h_hbm, src_hbm, dst_hbm, acc_out, acc_s,
             srcv_a, dstv_a, rows_a, sem_a, scsem_a,
             srcv_b, dstv_b, rows_b, sem_b, scsem_b):
        cid, sid, wid = _tile_ids()
        srcv = (srcv_a, srcv_b)
        dstv = (dstv_a, dstv_b)
        rows = (rows_a, rows_b)
        sem = (sem_a, sem_b)
        scsem = (scsem_a, scsem_b)

        def _zrow(i, carry):
            for j in range(D // 16):
                rows_a[i, pl.ds(j * 16, 16)] = jnp.zeros((16,), jnp.float32)
            return carry
        lax.fori_loop(0, _C, _zrow, 0)
        _zero_acc(acc_s, rows_a, sid)
        plsc.subcore_barrier()

        def _c(i):
            return i * _NW + wid

        def _drain(s):
            pltpu.make_async_copy(rows[s], acc_s.at[dstv[s]],
                                  scsem[s]).wait()

        def _start(chunk_idx, s, drain):
            if drain:
                _drain(s)
            base = chunk_idx * _C
            pltpu.sync_copy(src_hbm.at[pl.ds(base, _C)], srcv[s])
            pltpu.sync_copy(dst_hbm.at[pl.ds(base, _C)], dstv[s])
            pltpu.async_copy(h_hbm.at[srcv[s]], rows[s], sem[s])

        def _finish(chunk_idx, s):
            pltpu.make_async_copy(h_hbm.at[srcv[s]], rows[s], sem[s]).wait()
            pltpu.async_copy(rows[s], acc_s.at[dstv[s]], scsem[s], add=True)

        _start(_c(0), 0, False)
        _start(_c(1), 1, False)
        _finish(_c(0), 0)

        def _body(j, carry):
            k = 2 * j + 2
            _start(_c(k), 0, True)
            _finish(_c(k - 1), 1)
            _start(_c(k + 1), 1, True)
            _finish(_c(k), 0)
            return carry
        lax.fori_loop(0, (_PW - 3) // 2, _body, 0)
        _start(_c(_PW - 1), 0, True)
        _finish(_c(_PW - 2), 1)
        _finish(_c(_PW - 1), 0)
        _drain(1)
        _drain(0)

        plsc.subcore_barrier()
        _write_acc(acc_s, rows_a, acc_out, cid, sid)

    return pl.kernel(
        body,
        out_type=[jax.ShapeDtypeStruct((_NC, N, D), jnp.float32)],
        mesh=mesh,
        scratch_types=[
            pltpu.VMEM_SHARED((N, D), jnp.float32),
            pltpu.VMEM((_C,), jnp.int32),
            pltpu.VMEM((_C,), jnp.int32),
            pltpu.VMEM((_C, D), jnp.float32),
            pltpu.SemaphoreType.DMA,
            pltpu.SemaphoreType.DMA,
            pltpu.VMEM((_C,), jnp.int32),
            pltpu.VMEM((_C,), jnp.int32),
            pltpu.VMEM((_C, D), jnp.float32),
            pltpu.SemaphoreType.DMA,
            pltpu.SemaphoreType.DMA,
        ])


@functools.cache
def _make_sc_degree():
    mesh = plsc.VectorSubcoreMesh(core_axis_name="c", subcore_axis_name="s")

    def body(dst_hbm, deg_out, deg_s, ones_v, stage_v,
             dstv_a, sem_a, dstv_b, sem_b):
        cid, sid, wid = _tile_ids()
        dstv = (dstv_a, dstv_b)
        sem = (sem_a, sem_b)

        def _fill(i, carry):
            for j in range(D // 16):
                stage_v[i, pl.ds(j * 16, 16)] = jnp.zeros((16,), jnp.float32)
                ones_v[i, pl.ds(j * 16, 16)] = jnp.ones((16,), jnp.float32)
            return carry
        lax.fori_loop(0, _C, _fill, 0)
        _zero_acc(deg_s, stage_v, sid)
        plsc.subcore_barrier()

        def _start(chunk_idx, s):
            pltpu.async_copy(dst_hbm.at[pl.ds(chunk_idx * _C, _C)],
                             dstv[s], sem[s])

        def _finish(chunk_idx, s):
            pltpu.make_async_copy(dst_hbm.at[pl.ds(chunk_idx * _C, _C)],
                                  dstv[s], sem[s]).wait()
            pltpu.sync_copy(ones_v, deg_s.at[dstv[s]], add=True)

        _pipeline(wid, _start, _finish)
        plsc.subcore_barrier()
        _write_acc(deg_s, stage_v, deg_out, cid, sid)

    return pl.kernel(
        body,
        out_type=[jax.ShapeDtypeStruct((_NC, N, D), jnp.float32)],
        mesh=mesh,
        scratch_types=[
            pltpu.VMEM_SHARED((N, D), jnp.float32),
            pltpu.VMEM((_C, D), jnp.float32),
            pltpu.VMEM((_C, D), jnp.float32),
            pltpu.VMEM((_C,), jnp.int32),
            pltpu.SemaphoreType.DMA,
            pltpu.VMEM((_C,), jnp.int32),
            pltpu.SemaphoreType.DMA,
        ])


_BR = 1000


def _dense_body(relu, acc_ref, deg_ref, h_ref, wl_ref, wr_ref, bl_ref, o_ref):
    a = acc_ref[0] + acc_ref[1]
    dsum = deg_ref[0] + deg_ref[1]
    inv = 1.0 / jnp.maximum(dsum[:, 0:1], 1.0)
    out = (jnp.dot(a * inv, wl_ref[...], preferred_element_type=jnp.float32)
           + jnp.dot(h_ref[...], wr_ref[...], preferred_element_type=jnp.float32)
           + bl_ref[...])
    if relu:
        out = jnp.maximum(out, 0.0)
    o_ref[...] = out


def _dense(acc, deg, h, wlT, wrT, bl, relu):
    return pl.pallas_call(
        functools.partial(_dense_body, relu),
        out_shape=jax.ShapeDtypeStruct((N, D), jnp.float32),
        grid=(N // _BR,),
        in_specs=[
            pl.BlockSpec((_NC, _BR, D), lambda i: (0, i, 0)),
            pl.BlockSpec((_NC, _BR, D), lambda i: (0, i, 0)),
            pl.BlockSpec((_BR, D), lambda i: (i, 0)),
            pl.BlockSpec((D, D), lambda i: (0, 0)),
            pl.BlockSpec((D, D), lambda i: (0, 0)),
            pl.BlockSpec((1, D), lambda i: (0, 0)),
        ],
        out_specs=pl.BlockSpec((_BR, D), lambda i: (i, 0)),
    )(acc, deg, h, wlT, wrT, bl.reshape(1, D))


def kernel(x, edge_index, Wl1, bl1, Wr1, Wl2, bl2, Wr2):
    src = edge_index[0]
    dst = edge_index[1]
    (deg,) = _make_sc_degree()(dst)
    (acc1,) = _make_sc_aggregate()(x, src, dst)
    h1 = _dense(acc1, deg, x, Wl1.T, Wr1.T, bl1, relu=True)
    (acc2,) = _make_sc_aggregate()(h1, src, dst)
    logits = _dense(acc2, deg, h1, Wl2.T, Wr2.T, bl2, relu=False)
    return (h1, logits)


# speedup vs baseline: 1.3119x; 1.3119x over previous
import functools

import jax
import jax.numpy as jnp
from jax import lax
from jax.experimental import pallas as pl
from jax.experimental.pallas import tpu as pltpu
from jax.experimental.pallas import tpu_sc as plsc

N = 10000
D = 128
E = 320000

_NC = 2
_NS = 16
_NW = _NC * _NS
_C = 80
_NCHUNKS = E // _C
_PW = _NCHUNKS // _NW
_RPT = 624
_WB = 48


def _tile_ids():
    cid = lax.axis_index("c")
    sid = lax.axis_index("s")
    return cid, sid, sid * _NC + cid


def _zero_acc(acc_s, zsrc_v, sid):
    r0 = 16 + sid * _RPT
    for k in range(_RPT // _WB):
        pltpu.sync_copy(zsrc_v.at[pl.ds(0, _WB)],
                        acc_s.at[pl.ds(r0 + k * _WB, _WB)])

    @pl.when(sid < 2)
    def _zero_head():
        pltpu.sync_copy(zsrc_v.at[pl.ds(0, 8)], acc_s.at[pl.ds(sid * 8, 8)])


def _write_acc(acc_s, stage_v, out_hbm, cid, sid):
    r0 = 16 + sid * _RPT
    for k in range(_RPT // _WB):
        rs = r0 + k * _WB
        pltpu.sync_copy(acc_s.at[pl.ds(rs, _WB)], stage_v.at[pl.ds(0, _WB)])
        pltpu.sync_copy(stage_v.at[pl.ds(0, _WB)],
                        out_hbm.at[cid, pl.ds(rs, _WB)])

    @pl.when(sid < 2)
    def _write_head():
        hs = sid * 8
        pltpu.sync_copy(acc_s.at[pl.ds(hs, 8)], stage_v.at[pl.ds(0, 8)])
        pltpu.sync_copy(stage_v.at[pl.ds(0, 8)], out_hbm.at[cid, pl.ds(hs, 8)])


def _pipeline(wid, start, finish):
    def _c(i):
        return i * _NW + wid

    start(_c(0), 0)

    def _body(k, carry):
        start(_c(2 * k + 1), 1)
        finish(_c(2 * k), 0)
        start(_c(2 * k + 2), 0)
        finish(_c(2 * k + 1), 1)
        return carry
    lax.fori_loop(0, (_PW - 1) // 2, _body, 0)
    finish(_c(_PW - 1), 0)


@functools.cache
def _make_sc_aggregate():
    mesh = plsc.VectorSubcoreMesh(core_axis_name="c", subcore_axis_name="s")

    def body(h_hbm, e2_hbm, acc_out, acc_s,
             srcv_a, dstv_a, rows_a, sem_a, scsem_a, idxv_a, isem_a,
             srcv_b, dstv_b, rows_b, sem_b, scsem_b, idxv_b, isem_b):
        cid, sid, wid = _tile_ids()
        srcv = (srcv_a, srcv_b)
        dstv = (dstv_a, dstv_b)
        rows = (rows_a, rows_b)
        sem = (sem_a, sem_b)
        scsem = (scsem_a, scsem_b)
        idxv = (idxv_a, idxv_b)
        isem = (isem_a, isem_b)

        def _zrow(i, carry):
            for j in range(D // 16):
                rows_a[i, pl.ds(j * 16, 16)] = jnp.zeros((16,), jnp.float32)
            return carry
        lax.fori_loop(0, _C, _zrow, 0)
        _zero_acc(acc_s, rows_a, sid)
        plsc.subcore_barrier()

        def _c(i):
            return i * _NW + wid

        def _idx_load(chunk_idx, s):
            pltpu.async_copy(e2_hbm.at[pl.ds(chunk_idx * (2 * _C), 2 * _C)],
                             idxv[s], isem[s])

        def _drain(s):
            pltpu.make_async_copy(rows[s], acc_s.at[dstv[s]],
                                  scsem[s]).wait()

        def _start(chunk_idx, s, drain, prefetch_idx):
            if drain:
                _drain(s)
            pltpu.make_async_copy(
                e2_hbm.at[pl.ds(chunk_idx * (2 * _C), 2 * _C)],
                idxv[s], isem[s]).wait()
            for j in range(_C // 16):
                srcv[s][pl.ds(j * 16, 16)] = idxv[s][pl.ds(j * 16, 16)]
                dstv[s][pl.ds(j * 16, 16)] = idxv[s][pl.ds(_C + j * 16, 16)]
            if prefetch_idx is not None:
                _idx_load(prefetch_idx, s)
            pltpu.async_copy(h_hbm.at[srcv[s]], rows[s], sem[s])

        def _finish(chunk_idx, s):
            pltpu.make_async_copy(h_hbm.at[srcv[s]], rows[s], sem[s]).wait()
            pltpu.async_copy(rows[s], acc_s.at[dstv[s]], scsem[s], add=True)

        _idx_load(_c(0), 0)
        _idx_load(_c(1), 1)
        _start(_c(0), 0, False, _c(2))
        _start(_c(1), 1, False, _c(3))
        _finish(_c(0), 0)

        def _body(j, carry):
            k = 2 * j + 2
            _start(_c(k), 0, True, _c(k + 2))
            _finish(_c(k - 1), 1)
            _start(_c(k + 1), 1, True, _c(k + 3))
            _finish(_c(k), 0)
            return carry
        lax.fori_loop(0, (_PW - 5) // 2, _body, 0)
        _start(_c(_PW - 3), 0, True, _c(_PW - 1))
        _finish(_c(_PW - 4), 1)
        _start(_c(_PW - 2), 1, True, None)
        _finish(_c(_PW - 3), 0)
        _start(_c(_PW - 1), 0, True, None)
        _finish(_c(_PW - 2), 1)
        _finish(_c(_PW - 1), 0)
        _drain(1)
        _drain(0)

        plsc.subcore_barrier()
        _write_acc(acc_s, rows_a, acc_out, cid, sid)

    return pl.kernel(
        body,
        out_type=[jax.ShapeDtypeStruct((_NC, N, D), jnp.float32)],
        mesh=mesh,
        scratch_types=[
            pltpu.VMEM_SHARED((N, D), jnp.float32),
            pltpu.VMEM((_C,), jnp.int32),
            pltpu.VMEM((_C,), jnp.int32),
            pltpu.VMEM((_C, D), jnp.float32),
            pltpu.SemaphoreType.DMA,
            pltpu.SemaphoreType.DMA,
            pltpu.VMEM((2 * _C,), jnp.int32),
            pltpu.SemaphoreType.DMA,
            pltpu.VMEM((_C,), jnp.int32),
            pltpu.VMEM((_C,), jnp.int32),
            pltpu.VMEM((_C, D), jnp.float32),
            pltpu.SemaphoreType.DMA,
            pltpu.SemaphoreType.DMA,
            pltpu.VMEM((2 * _C,), jnp.int32),
            pltpu.SemaphoreType.DMA,
        ])


@functools.cache
def _make_sc_degree():
    mesh = plsc.VectorSubcoreMesh(core_axis_name="c", subcore_axis_name="s")

    def body(dst_hbm, deg_out, deg_s, ones_v, stage_v,
             dstv_a, sem_a, dstv_b, sem_b):
        cid, sid, wid = _tile_ids()
        dstv = (dstv_a, dstv_b)
        sem = (sem_a, sem_b)

        def _fill(i, carry):
            for j in range(D // 16):
                stage_v[i, pl.ds(j * 16, 16)] = jnp.zeros((16,), jnp.float32)
                ones_v[i, pl.ds(j * 16, 16)] = jnp.ones((16,), jnp.float32)
            return carry
        lax.fori_loop(0, _C, _fill, 0)
        _zero_acc(deg_s, stage_v, sid)
        plsc.subcore_barrier()

        def _start(chunk_idx, s):
            pltpu.async_copy(dst_hbm.at[pl.ds(chunk_idx * _C, _C)],
                             dstv[s], sem[s])

        def _finish(chunk_idx, s):
            pltpu.make_async_copy(dst_hbm.at[pl.ds(chunk_idx * _C, _C)],
                                  dstv[s], sem[s]).wait()
            pltpu.sync_copy(ones_v, deg_s.at[dstv[s]], add=True)

        _pipeline(wid, _start, _finish)
        plsc.subcore_barrier()
        _write_acc(deg_s, stage_v, deg_out, cid, sid)

    return pl.kernel(
        body,
        out_type=[jax.ShapeDtypeStruct((_NC, N, D), jnp.float32)],
        mesh=mesh,
        scratch_types=[
            pltpu.VMEM_SHARED((N, D), jnp.float32),
            pltpu.VMEM((_C, D), jnp.float32),
            pltpu.VMEM((_C, D), jnp.float32),
            pltpu.VMEM((_C,), jnp.int32),
            pltpu.SemaphoreType.DMA,
            pltpu.VMEM((_C,), jnp.int32),
            pltpu.SemaphoreType.DMA,
        ])


_BR = 1000


def _dense_body(relu, acc_ref, deg_ref, h_ref, wl_ref, wr_ref, bl_ref, o_ref):
    a = acc_ref[0] + acc_ref[1]
    dsum = deg_ref[0] + deg_ref[1]
    inv = 1.0 / jnp.maximum(dsum[:, 0:1], 1.0)
    out = (jnp.dot(a * inv, wl_ref[...], preferred_element_type=jnp.float32)
           + jnp.dot(h_ref[...], wr_ref[...], preferred_element_type=jnp.float32)
           + bl_ref[...])
    if relu:
        out = jnp.maximum(out, 0.0)
    o_ref[...] = out


def _dense(acc, deg, h, wlT, wrT, bl, relu):
    return pl.pallas_call(
        functools.partial(_dense_body, relu),
        out_shape=jax.ShapeDtypeStruct((N, D), jnp.float32),
        grid=(N // _BR,),
        in_specs=[
            pl.BlockSpec((_NC, _BR, D), lambda i: (0, i, 0)),
            pl.BlockSpec((_NC, _BR, D), lambda i: (0, i, 0)),
            pl.BlockSpec((_BR, D), lambda i: (i, 0)),
            pl.BlockSpec((D, D), lambda i: (0, 0)),
            pl.BlockSpec((D, D), lambda i: (0, 0)),
            pl.BlockSpec((1, D), lambda i: (0, 0)),
        ],
        out_specs=pl.BlockSpec((_BR, D), lambda i: (i, 0)),
    )(acc, deg, h, wlT, wrT, bl.reshape(1, D))


def kernel(x, edge_index, Wl1, bl1, Wr1, Wl2, bl2, Wr2):
    src = edge_index[0]
    dst = edge_index[1]
    e2 = jnp.concatenate(
        [src.reshape(_NCHUNKS, _C), dst.reshape(_NCHUNKS, _C)],
        axis=1).reshape(-1)
    (deg,) = _make_sc_degree()(dst)
    (acc1,) = _make_sc_aggregate()(x, e2)
    h1 = _dense(acc1, deg, x, Wl1.T, Wr1.T, bl1, relu=True)
    (acc2,) = _make_sc_aggregate()(h1, e2)
    logits = _dense(acc2, deg, h1, Wl2.T, Wr2.T, bl2, relu=False)
    return (h1, logits)
